# baseline (speedup 1.0000x reference)
import jax
import jax.numpy as jnp
from jax.experimental import pallas as pl
import jax.experimental.pallas.tpu as pltpu


def _swap_lane_bits(v, lam, a, b):
    d = (1 << a) - (1 << b)
    ba = (lam >> a) & 1
    bb = (lam >> b) & 1
    vp = pltpu.roll(v, d, axis=1)
    vm = pltpu.roll(v, 128 - d, axis=1)
    return jnp.where(ba == bb, v, jnp.where(ba == 1, vp, vm))


def _swap_col_lane_bit(lo, hi, lam, lane_bit):
    d = 1 << lane_bit
    bl = (lam >> lane_bit) & 1
    new_lo = jnp.where(bl == 1, pltpu.roll(hi, d, axis=1), lo)
    new_hi = jnp.where(bl == 0, pltpu.roll(lo, 128 - d, axis=1), hi)
    return new_lo, new_hi


def _permute_body(x_ref, o_ref):
    bm = x_ref.shape[0]
    lam = jax.lax.broadcasted_iota(jnp.int32, (bm, 128), 1)
    t = [x_ref[:, 128 * k:128 * (k + 1)] for k in range(8)]
    t = [_swap_lane_bits(v, lam, 4, 0) for v in t]
    t = [_swap_lane_bits(v, lam, 3, 1) for v in t]
    for k in (0, 1, 2, 3):
        t[k], t[k + 4] = _swap_col_lane_bit(t[k], t[k + 4], lam, 5)
    for k in (0, 1, 4, 5):
        t[k], t[k + 2] = _swap_col_lane_bit(t[k], t[k + 2], lam, 6)
    for k in range(8):
        o_ref[:, 128 * k:128 * (k + 1)] = t[k]


def kernel(tensor):
    n, r, c = tensor.shape
    xf = tensor.reshape(n, r * c)
    bm = 512
    out = pl.pallas_call(
        _permute_body,
        grid=(n // bm,),
        in_specs=[pl.BlockSpec((bm, r * c), lambda i: (i, 0))],
        out_specs=pl.BlockSpec((bm, r * c), lambda i: (i, 0)),
        out_shape=jax.ShapeDtypeStruct((n, r * c), tensor.dtype),
        compiler_params=pltpu.CompilerParams(
            dimension_semantics=("parallel",)),
    )(xf)
    return out.reshape(n, r, c)
